# baseline (speedup 1.0000x reference)
import jax
import jax.numpy as jnp
from jax import lax
from jax.experimental import pallas as pl
from jax.experimental.pallas import tpu as pltpu
from jax.experimental.pallas import tpu_sc as plsc

B = 16384
F = 100
V = 1000000
NC = 2
NS = 16
NW = NC * NS
ROWS_PER_W = B // NW
RBLK = 128
NBLK = ROWS_PER_W // RBLK
EBLK = RBLK * F
NGATH = EBLK // 128


def _row_accumulate(gat_v, val_v, par, base):
    lane = jax.lax.iota(jnp.int32, 16)
    zero = jnp.zeros((16,), jnp.float32)
    accs = [zero, zero, zero, zero]
    for j in range(25):
        v = (gat_v[par, pl.ds(base + j * 16, 16)]
             * val_v[par, pl.ds(base + j * 16, 16)])
        e0 = j * 16
        r0 = e0 // F
        r1 = (e0 + 15) // F
        if r0 == r1:
            accs[r0] = accs[r0] + v
        else:
            cut = r1 * F - e0
            accs[r0] = accs[r0] + jnp.where(lane < cut, v, 0.0)
            accs[r1] = accs[r1] + jnp.where(lane < cut, 0.0, v)
    return accs


def _sc_body(idx_hbm, val_hbm, w_hbm, bias_hbm, out_hbm,
             idx_v, val_v, gat_v, out_v, bias_v, drain_v,
             sem_in0, sem_in1, sem_g0, sem_g1):
    wid = lax.axis_index("s") * NC + lax.axis_index("c")
    ebase_w = wid * NBLK * EBLK
    row_base = wid * ROWS_PER_W
    pltpu.sync_copy(bias_hbm, bias_v)
    lane = jax.lax.iota(jnp.int32, 16)
    sem_in = (sem_in0, sem_in1)
    sem_g = (sem_g0, sem_g1)

    def start_in(blk):
        par = blk & 1
        ebase = ebase_w + blk * EBLK
        hi = pltpu.async_copy(idx_hbm.at[pl.ds(ebase, EBLK)],
                              idx_v.at[par], sem_in[par])
        hv = pltpu.async_copy(val_hbm.at[pl.ds(ebase, EBLK)],
                              val_v.at[par], sem_in[par])
        return hi, hv

    def issue_gathers(blk):
        par = blk & 1

        def issue(j, c):
            pltpu.async_copy(
                w_hbm.at[0].at[idx_v.at[par].at[pl.ds(j * 128, 128)]],
                gat_v.at[par].at[pl.ds(j * 128, 128)], sem_g[par])
            return c

        lax.fori_loop(0, NGATH, issue, 0)

    def drain_gathers(blk):
        pltpu.make_async_copy(w_hbm.at[0].at[pl.ds(0, EBLK)],
                              drain_v, sem_g[blk & 1]).wait()

    def compute(blk):
        par = blk & 1
        row0 = row_base + blk * RBLK
        bvec = bias_v[...]

        def sixteen_rows(rg, c):
            gbase = rg * (16 * F)
            outv = jnp.zeros((16,), jnp.float32)
            for u in range(4):
                accs = _row_accumulate(gat_v, val_v, par, gbase + u * 400)
                for k in range(4):
                    s = jnp.sum(accs[k])
                    outv = jnp.where(lane == (u * 4 + k), s, outv)
            out_v[pl.ds(rg * 16, 16)] = outv + bvec
            return c

        lax.fori_loop(0, RBLK // 16, sixteen_rows, 0)
        pltpu.sync_copy(out_v, out_hbm.at[pl.ds(row0, RBLK)])

    handles = {0: start_in(0)}
    handles[0][0].wait()
    handles[0][1].wait()
    issue_gathers(0)
    handles[1] = start_in(1)
    for blk in range(NBLK):
        if blk + 1 < NBLK:
            handles[blk + 1][0].wait()
            handles[blk + 1][1].wait()
            issue_gathers(blk + 1)
        drain_gathers(blk)
        compute(blk)
        if blk + 2 < NBLK:
            handles[blk + 2] = start_in(blk + 2)


@jax.jit
def _sc_call(idx_flat, val_flat, w2d, bias16):
    mesh = plsc.VectorSubcoreMesh(core_axis_name="c", subcore_axis_name="s")
    f = pl.kernel(
        _sc_body,
        mesh=mesh,
        out_type=jax.ShapeDtypeStruct((B,), jnp.float32),
        scratch_types=[
            pltpu.VMEM((2, EBLK), jnp.int32),
            pltpu.VMEM((2, EBLK), jnp.float32),
            pltpu.VMEM((2, EBLK), jnp.float32),
            pltpu.VMEM((RBLK,), jnp.float32),
            pltpu.VMEM((16,), jnp.float32),
            pltpu.VMEM((EBLK,), jnp.float32),
            pltpu.SemaphoreType.DMA,
            pltpu.SemaphoreType.DMA,
            pltpu.SemaphoreType.DMA,
            pltpu.SemaphoreType.DMA,
        ],
        compiler_params=pltpu.CompilerParams(needs_layout_passes=False),
    )
    return f(idx_flat, val_flat, w2d, bias16)


def kernel(index_list, value_list, W, bias):
    idx_flat = index_list.reshape(B * F)
    val_flat = value_list.reshape(B * F)
    bias16 = jnp.broadcast_to(bias, (16,))
    res = _sc_call(idx_flat, val_flat, W, bias16)
    return res.reshape(B, 1)
